# baseline (speedup 1.0000x reference)
import functools

import jax
import jax.numpy as jnp
from jax import lax
from jax.experimental import pallas as pl
from jax.experimental.pallas import tpu as pltpu
from jax.experimental.pallas import tpu_sc as plsc

N = 10000
NP = 10240
H = 256
HQ = 64
M = 160000
CS = 128
CH = 79
MP = 16 * CH * CS
RPT = NP // 16


def _u_body(h_ref, w_ref, b_ref, o_ref):
    p = jnp.dot(h_ref[...], w_ref[...], preferred_element_type=jnp.float32, precision=lax.Precision.HIGHEST)
    p = p + b_ref[...]
    o_ref[...] = jnp.stack([p[:, 0:64], p[:, 64:128],
                            p[:, 128:192], p[:, 192:256]], axis=0)


def _e_body(a_ref, w_ref, o_ref):
    p = jnp.dot(a_ref[...], w_ref[...], preferred_element_type=jnp.float32, precision=lax.Precision.HIGHEST)
    o_ref[...] = jnp.stack([p[:, 0:64], p[:, 64:128],
                            p[:, 128:192], p[:, 192:256]], axis=0)


def _ep_body(r_ref, d_ref, h_ref, w2_ref, b2_ref, w3a_ref, w3b_ref, b3_ref,
             w4_ref, b4_ref, g_ref, be_ref, o_ref):
    rb = jnp.concatenate([r_ref[0], r_ref[1], r_ref[2], r_ref[3]], axis=1)
    deg = d_ref[0][:, 0:1]
    agg = jnp.dot(rb, w2_ref[...], preferred_element_type=jnp.float32, precision=lax.Precision.HIGHEST)
    agg = agg + deg * b2_ref[...]
    z = jnp.dot(h_ref[...], w3a_ref[...], preferred_element_type=jnp.float32, precision=lax.Precision.HIGHEST)
    z = z + jnp.dot(agg, w3b_ref[...], preferred_element_type=jnp.float32, precision=lax.Precision.HIGHEST)
    z = z + b3_ref[...]
    a1 = jnp.maximum(z, 0.0)
    dh = jnp.dot(a1, w4_ref[...], preferred_element_type=jnp.float32, precision=lax.Precision.HIGHEST)
    dh = dh + b4_ref[...]
    y = h_ref[...] + dh
    mu = jnp.mean(y, axis=1, keepdims=True)
    d0 = y - mu
    var = jnp.mean(d0 * d0, axis=1, keepdims=True)
    o_ref[...] = d0 * lax.rsqrt(var + 1e-5) * g_ref[...] + be_ref[...]


def _sc_body(u_hbm, e_hbm, src_hbm, dst_hbm, r_hbm, d_hbm,
             sidx, didx, ubuf, ebuf, obuf, zbuf, acc, dacc, sem):
    c = lax.axis_index("c")
    s = lax.axis_index("s")

    def _init_row(i, _):
        obuf[i, :] = jnp.ones((16,), jnp.float32)
        zbuf[i, :] = jnp.zeros((16,), jnp.float32)
        return 0
    lax.fori_loop(0, CS, _init_row, 0)

    for phase in range(2):
        q = c * 2 + phase

        def _zero_row(i, _):
            def _zero_col(j, _):
                ubuf[i, pl.ds(j * 16, 16)] = jnp.zeros((16,), jnp.float32)
                return 0
            lax.fori_loop(0, HQ // 16, _zero_col, 0)
            return 0
        lax.fori_loop(0, CS, _zero_row, 0)
        for k in range(RPT // CS):
            pltpu.sync_copy(ubuf, acc.at[pl.ds(s * RPT + k * CS, CS), :])
            if phase == 0:
                pltpu.sync_copy(zbuf,
                                dacc.at[pl.ds(s * RPT + k * CS, CS), :])
        plsc.subcore_barrier()

        def _chunk(t, _):
            base = (s * CH + t) * CS
            pltpu.sync_copy(src_hbm.at[pl.ds(base, CS)], sidx)
            pltpu.sync_copy(dst_hbm.at[pl.ds(base, CS)], didx)
            off = q * NP

            def _addoff(j, _):
                sidx[pl.ds(j * 16, 16)] = sidx[pl.ds(j * 16, 16)] + off
                return 0
            lax.fori_loop(0, CS // 16, _addoff, 0)

            pltpu.async_copy(u_hbm.at[sidx], ubuf, sem).wait()
            pltpu.sync_copy(e_hbm.at[pl.ds(q * MP + base, CS), :], ebuf)

            def _row(i, _):
                def _col(j, _):
                    x = ubuf[i, pl.ds(j * 16, 16)] + ebuf[i, pl.ds(j * 16, 16)]
                    ubuf[i, pl.ds(j * 16, 16)] = jnp.maximum(x, 0.0)
                    return 0
                lax.fori_loop(0, HQ // 16, _col, 0)
                return 0
            lax.fori_loop(0, CS, _row, 0)

            pltpu.sync_copy(ubuf, acc.at[didx], add=True)
            if phase == 0:
                pltpu.sync_copy(obuf, dacc.at[didx], add=True)
            return 0
        lax.fori_loop(0, CH, _chunk, 0)
        plsc.subcore_barrier()

        pltpu.sync_copy(acc.at[pl.ds(s * RPT, RPT), :],
                        r_hbm.at[q, pl.ds(s * RPT, RPT), :])
        if phase == 0:
            pltpu.sync_copy(dacc.at[pl.ds(s * RPT, RPT), :],
                            d_hbm.at[c, pl.ds(s * RPT, RPT), :])


def kernel(h, edge_index, edge_attr, W1, b1, W2, b2, W3, b3, W4, b4,
           gamma, beta):
    src = edge_index[0].astype(jnp.int32)
    dst = edge_index[1].astype(jnp.int32)
    pad = MP - M
    srcp = jnp.concatenate([src, jnp.full((pad,), N, jnp.int32)])
    dstp = jnp.concatenate([dst, jnp.full((pad,), N, jnp.int32)])
    h_pad = jnp.concatenate([h, jnp.zeros((NP - N, H), jnp.float32)], axis=0)
    ea_pad = jnp.concatenate(
        [edge_attr, jnp.zeros((pad, edge_attr.shape[1]), jnp.float32)], axis=0)

    u = pl.pallas_call(
        _u_body,
        grid=(NP // 128,),
        in_specs=[pl.BlockSpec((128, H), lambda i: (i, 0)),
                  pl.BlockSpec((H, H), lambda i: (0, 0)),
                  pl.BlockSpec((1, H), lambda i: (0, 0))],
        out_specs=pl.BlockSpec((4, 128, HQ), lambda i: (0, i, 0)),
        out_shape=jax.ShapeDtypeStruct((4, NP, HQ), jnp.float32),
    )(h_pad, W1[:H], b1.reshape(1, H))
    u_flat = u.reshape(4 * NP, HQ)

    EB = 512
    e = pl.pallas_call(
        _e_body,
        grid=(MP // EB,),
        in_specs=[pl.BlockSpec((EB, 16), lambda i: (i, 0)),
                  pl.BlockSpec((16, H), lambda i: (0, 0))],
        out_specs=pl.BlockSpec((4, EB, HQ), lambda i: (0, i, 0)),
        out_shape=jax.ShapeDtypeStruct((4, MP, HQ), jnp.float32),
    )(ea_pad, W1[H:])
    e_flat = e.reshape(4 * MP, HQ)

    mesh = plsc.VectorSubcoreMesh(core_axis_name="c", subcore_axis_name="s")
    r, d = pl.kernel(
        _sc_body,
        mesh=mesh,
        compiler_params=pltpu.CompilerParams(use_tc_tiling_on_sc=False),
        out_type=[jax.ShapeDtypeStruct((4, NP, HQ), jnp.float32),
                  jax.ShapeDtypeStruct((2, NP, 16), jnp.float32)],
        scratch_types=[
            pltpu.VMEM((CS,), jnp.int32),
            pltpu.VMEM((CS,), jnp.int32),
            pltpu.VMEM((CS, HQ), jnp.float32),
            pltpu.VMEM((CS, HQ), jnp.float32),
            pltpu.VMEM((CS, 16), jnp.float32),
            pltpu.VMEM((CS, 16), jnp.float32),
            pltpu.VMEM_SHARED((NP, HQ), jnp.float32),
            pltpu.VMEM_SHARED((NP, 16), jnp.float32),
            pltpu.SemaphoreType.DMA,
        ],
    )(u_flat, e_flat, srcp, dstp)

    out = pl.pallas_call(
        _ep_body,
        grid=(NP // 128,),
        in_specs=[
            pl.BlockSpec((4, 128, HQ), lambda i: (0, i, 0)),
            pl.BlockSpec((2, 128, 16), lambda i: (0, i, 0)),
            pl.BlockSpec((128, H), lambda i: (i, 0)),
            pl.BlockSpec((H, H), lambda i: (0, 0)),
            pl.BlockSpec((1, H), lambda i: (0, 0)),
            pl.BlockSpec((H, H), lambda i: (0, 0)),
            pl.BlockSpec((H, H), lambda i: (0, 0)),
            pl.BlockSpec((1, H), lambda i: (0, 0)),
            pl.BlockSpec((H, H), lambda i: (0, 0)),
            pl.BlockSpec((1, H), lambda i: (0, 0)),
            pl.BlockSpec((1, H), lambda i: (0, 0)),
            pl.BlockSpec((1, H), lambda i: (0, 0)),
        ],
        out_specs=pl.BlockSpec((128, H), lambda i: (i, 0)),
        out_shape=jax.ShapeDtypeStruct((NP, H), jnp.float32),
    )(r, d, h_pad, W2, b2.reshape(1, H), W3[:H], W3[H:], b3.reshape(1, H),
      W4, b4.reshape(1, H), gamma.reshape(1, H), beta.reshape(1, H))
    return out[:N]
